# baseline (speedup 1.0000x reference)
import functools

import jax
import jax.numpy as jnp
from jax.experimental import pallas as pl
from jax.experimental.pallas import tpu as pltpu

_P = 512
_M = 2
_DB = 256
_D = _M * _DB
_TILE = 392


def _vq_kernel(q_ref, mem_ref, emb_ref, idx_ref, loss_ref,
               wstack_ref, mnorm_ref, mhi_ref, mlo_ref, *, inv_count):
    step = pl.program_id(0)
    nsteps = pl.num_programs(0)

    @pl.when(step == 0)
    def _init():
        m = mem_ref[...]
        wdt2 = jnp.swapaxes(m, 0, 1) * -2.0
        mnorm_ref[...] = 0.25 * jnp.sum(wdt2 * wdt2, axis=0,
                                        keepdims=True)
        wdh = wdt2.astype(jnp.bfloat16)
        wdl = (wdt2 - wdh.astype(jnp.float32)).astype(jnp.bfloat16)
        wstack_ref[0 * _DB:1 * _DB, :] = wdh
        wstack_ref[1 * _DB:2 * _DB, :] = wdh
        wstack_ref[2 * _DB:3 * _DB, :] = wdl
        mhi = m.astype(jnp.bfloat16)
        mhi_ref[...] = mhi
        mlo_ref[...] = (m - mhi.astype(jnp.float32)).astype(jnp.bfloat16)
        loss_ref[...] = jnp.zeros((1, 1), jnp.float32)

    q = q_ref[...]
    qh = q.astype(jnp.bfloat16)
    ql = (q - qh.astype(jnp.float32)).astype(jnp.bfloat16)
    rows = q.shape[0]
    iota = jax.lax.broadcasted_iota(jnp.int32, (rows, _P), 1)
    loss_acc = jnp.zeros((), jnp.float32)
    dn = (((1,), (0,)), ((), ()))
    for i in range(_M):
        qi = q[:, i * _DB:(i + 1) * _DB]
        qhi = qh[:, i * _DB:(i + 1) * _DB]
        qli = ql[:, i * _DB:(i + 1) * _DB]
        qcat = jnp.concatenate([qhi, qli, qhi], axis=1)
        scores2 = jax.lax.dot_general(
            qcat, wstack_ref[:, i * _P:(i + 1) * _P],
            dn, preferred_element_type=jnp.float32)
        dist = mnorm_ref[:, i * _P:(i + 1) * _P] + scores2
        rmin = jnp.min(dist, axis=1, keepdims=True)
        ismin = dist == rmin
        idx = jnp.min(jnp.where(ismin, iota, _P),
                      axis=1, keepdims=True)
        idx_ref[:, i:i + 1] = idx + i * _P
        onehot = jnp.logical_and(ismin, iota == idx).astype(jnp.bfloat16)
        gathered = jax.lax.dot_general(
            onehot, mhi_ref[i * _P:(i + 1) * _P, :],
            dn, preferred_element_type=jnp.float32)
        gathered = gathered + jax.lax.dot_general(
            onehot, mlo_ref[i * _P:(i + 1) * _P, :],
            dn, preferred_element_type=jnp.float32)
        diff = gathered - qi
        emb_ref[:, i * _DB:(i + 1) * _DB] = diff + qi
        loss_acc = loss_acc + jnp.sum(diff * diff)
    loss_ref[...] += jnp.reshape(loss_acc, (1, 1))

    @pl.when(step == nsteps - 1)
    def _final():
        loss_ref[...] *= inv_count


def kernel(queries, mem):
    B, N, D = queries.shape
    bn = B * N
    flat_q = queries.reshape(bn, D)
    grid = (bn // _TILE,)
    emb, idx, loss = pl.pallas_call(
        functools.partial(_vq_kernel, inv_count=1.0 / float(bn * D)),
        grid=grid,
        in_specs=[
            pl.BlockSpec((_TILE, D), lambda i: (i, 0)),
            pl.BlockSpec((_M * _P, _DB), lambda i: (0, 0)),
        ],
        out_specs=(
            pl.BlockSpec((_TILE, D), lambda i: (i, 0)),
            pl.BlockSpec((_TILE, _M), lambda i: (i, 0)),
            pl.BlockSpec((1, 1), lambda i: (0, 0)),
        ),
        out_shape=(
            jax.ShapeDtypeStruct((bn, D), jnp.float32),
            jax.ShapeDtypeStruct((bn, _M), jnp.int32),
            jax.ShapeDtypeStruct((1, 1), jnp.float32),
        ),
        scratch_shapes=[
            pltpu.VMEM((3 * _DB, _M * _P), jnp.bfloat16),
            pltpu.VMEM((1, _M * _P), jnp.float32),
            pltpu.VMEM((_M * _P, _DB), jnp.bfloat16),
            pltpu.VMEM((_M * _P, _DB), jnp.bfloat16),
        ],
    )(flat_q, mem)
    embeddings = emb.reshape(B, N, D)
    indices = idx.reshape(B, N, _M)
    vq_loss = jnp.zeros((), jnp.float32)
    commitment_loss = loss.reshape(())
    return (embeddings, indices, vq_loss, commitment_loss)


# speedup vs baseline: 1.0437x; 1.0437x over previous
import functools

import jax
import jax.numpy as jnp
from jax.experimental import pallas as pl
from jax.experimental.pallas import tpu as pltpu

_P = 512
_M = 2
_DB = 256
_D = _M * _DB
_TILE = 784


def _vq_kernel(q_ref, mem_ref, emb_ref, idx_ref, loss_ref,
               wstack_ref, mnorm_ref, mhi_ref, mlo_ref, *, inv_count):
    step = pl.program_id(0)
    nsteps = pl.num_programs(0)

    @pl.when(step == 0)
    def _init():
        m = mem_ref[...]
        wdt2 = jnp.swapaxes(m, 0, 1) * -2.0
        mnorm_ref[...] = 0.25 * jnp.sum(wdt2 * wdt2, axis=0,
                                        keepdims=True)
        wdh = wdt2.astype(jnp.bfloat16)
        wdl = (wdt2 - wdh.astype(jnp.float32)).astype(jnp.bfloat16)
        wstack_ref[0 * _DB:1 * _DB, :] = wdh
        wstack_ref[1 * _DB:2 * _DB, :] = wdh
        wstack_ref[2 * _DB:3 * _DB, :] = wdl
        mhi = m.astype(jnp.bfloat16)
        mhi_ref[...] = mhi
        mlo_ref[...] = (m - mhi.astype(jnp.float32)).astype(jnp.bfloat16)
        loss_ref[...] = jnp.zeros((1, 1), jnp.float32)

    q = q_ref[...]
    qh = q.astype(jnp.bfloat16)
    ql = (q - qh.astype(jnp.float32)).astype(jnp.bfloat16)
    rows = q.shape[0]
    iota = jax.lax.broadcasted_iota(jnp.int32, (rows, _P), 1)
    loss_acc = jnp.zeros((), jnp.float32)
    dn = (((1,), (0,)), ((), ()))
    for i in range(_M):
        qi = q[:, i * _DB:(i + 1) * _DB]
        qhi = qh[:, i * _DB:(i + 1) * _DB]
        qli = ql[:, i * _DB:(i + 1) * _DB]
        qcat = jnp.concatenate([qhi, qli, qhi], axis=1)
        scores2 = jax.lax.dot_general(
            qcat, wstack_ref[:, i * _P:(i + 1) * _P],
            dn, preferred_element_type=jnp.float32)
        dist = mnorm_ref[:, i * _P:(i + 1) * _P] + scores2
        rmin = jnp.min(dist, axis=1, keepdims=True)
        ismin = dist == rmin
        idx = jnp.min(jnp.where(ismin, iota, _P),
                      axis=1, keepdims=True)
        idx_ref[:, i:i + 1] = idx + i * _P
        onehot = jnp.logical_and(ismin, iota == idx).astype(jnp.bfloat16)
        gathered = jax.lax.dot_general(
            onehot, mhi_ref[i * _P:(i + 1) * _P, :],
            dn, preferred_element_type=jnp.float32)
        gathered = gathered + jax.lax.dot_general(
            onehot, mlo_ref[i * _P:(i + 1) * _P, :],
            dn, preferred_element_type=jnp.float32)
        diff = gathered - qi
        emb_ref[:, i * _DB:(i + 1) * _DB] = diff + qi
        loss_acc = loss_acc + jnp.sum(diff * diff)
    loss_ref[...] += jnp.reshape(loss_acc, (1, 1))

    @pl.when(step == nsteps - 1)
    def _final():
        loss_ref[...] *= inv_count


def kernel(queries, mem):
    B, N, D = queries.shape
    bn = B * N
    flat_q = queries.reshape(bn, D)
    grid = (bn // _TILE,)
    emb, idx, loss = pl.pallas_call(
        functools.partial(_vq_kernel, inv_count=1.0 / float(bn * D)),
        grid=grid,
        in_specs=[
            pl.BlockSpec((_TILE, D), lambda i: (i, 0)),
            pl.BlockSpec((_M * _P, _DB), lambda i: (0, 0)),
        ],
        out_specs=(
            pl.BlockSpec((_TILE, D), lambda i: (i, 0)),
            pl.BlockSpec((_TILE, _M), lambda i: (i, 0)),
            pl.BlockSpec((1, 1), lambda i: (0, 0)),
        ),
        out_shape=(
            jax.ShapeDtypeStruct((bn, D), jnp.float32),
            jax.ShapeDtypeStruct((bn, _M), jnp.int32),
            jax.ShapeDtypeStruct((1, 1), jnp.float32),
        ),
        scratch_shapes=[
            pltpu.VMEM((3 * _DB, _M * _P), jnp.bfloat16),
            pltpu.VMEM((1, _M * _P), jnp.float32),
            pltpu.VMEM((_M * _P, _DB), jnp.bfloat16),
            pltpu.VMEM((_M * _P, _DB), jnp.bfloat16),
        ],
    )(flat_q, mem)
    embeddings = emb.reshape(B, N, D)
    indices = idx.reshape(B, N, _M)
    vq_loss = jnp.zeros((), jnp.float32)
    commitment_loss = loss.reshape(())
    return (embeddings, indices, vq_loss, commitment_loss)


# speedup vs baseline: 1.0581x; 1.0138x over previous
import functools

import jax
import jax.numpy as jnp
from jax.experimental import pallas as pl
from jax.experimental.pallas import tpu as pltpu

_P = 512
_M = 2
_DB = 256
_D = _M * _DB
_TILE = 784


def _vq_kernel(q_ref, mem_ref, emb_ref, idx_ref, loss_ref,
               wstack_ref, mnorm_ref, mhi_ref, mlo_ref, *, inv_count):
    step = pl.program_id(0)
    nsteps = pl.num_programs(0)

    @pl.when(step == 0)
    def _init():
        m = mem_ref[...]
        wdt2 = jnp.swapaxes(m, 0, 1) * -2.0
        mnorm_ref[...] = 0.25 * jnp.sum(wdt2 * wdt2, axis=0,
                                        keepdims=True)
        wdh = wdt2.astype(jnp.bfloat16)
        wdl = (wdt2 - wdh.astype(jnp.float32)).astype(jnp.bfloat16)
        wstack_ref[0 * _DB:1 * _DB, :] = wdh
        wstack_ref[1 * _DB:2 * _DB, :] = wdh
        wstack_ref[2 * _DB:3 * _DB, :] = wdl
        mhi = m.astype(jnp.bfloat16)
        mhi_ref[...] = mhi
        mlo_ref[...] = (m - mhi.astype(jnp.float32)).astype(jnp.bfloat16)
        loss_ref[...] = jnp.zeros((1, 1), jnp.float32)

    q = q_ref[...]
    qh = q.astype(jnp.bfloat16)
    ql = (q - qh.astype(jnp.float32)).astype(jnp.bfloat16)
    rows = q.shape[0]
    iota = jax.lax.broadcasted_iota(jnp.int32, (rows, _P), 1).astype(jnp.float32)
    loss_acc = jnp.zeros((), jnp.float32)
    dn = (((1,), (0,)), ((), ()))
    for i in range(_M):
        qi = q[:, i * _DB:(i + 1) * _DB]
        qhi = qh[:, i * _DB:(i + 1) * _DB]
        qli = ql[:, i * _DB:(i + 1) * _DB]
        qcat = jnp.concatenate([qhi, qli, qhi], axis=1)
        scores2 = jax.lax.dot_general(
            qcat, wstack_ref[:, i * _P:(i + 1) * _P],
            dn, preferred_element_type=jnp.float32)
        dist = mnorm_ref[:, i * _P:(i + 1) * _P] + scores2
        rmin = jnp.min(dist, axis=1, keepdims=True)
        ismin = dist == rmin
        idxf = jnp.min(jnp.where(ismin, iota, float(_P)),
                       axis=1, keepdims=True)
        idx_ref[:, i:i + 1] = idxf.astype(jnp.int32) + i * _P
        onehot = jnp.logical_and(ismin, iota == idxf).astype(jnp.bfloat16)
        gathered = jax.lax.dot_general(
            onehot, mhi_ref[i * _P:(i + 1) * _P, :],
            dn, preferred_element_type=jnp.float32)
        gathered = gathered + jax.lax.dot_general(
            onehot, mlo_ref[i * _P:(i + 1) * _P, :],
            dn, preferred_element_type=jnp.float32)
        diff = gathered - qi
        emb_ref[:, i * _DB:(i + 1) * _DB] = diff + qi
        loss_acc = loss_acc + jnp.sum(diff * diff)
    loss_ref[...] += jnp.reshape(loss_acc, (1, 1))

    @pl.when(step == nsteps - 1)
    def _final():
        loss_ref[...] *= inv_count


def kernel(queries, mem):
    B, N, D = queries.shape
    bn = B * N
    flat_q = queries.reshape(bn, D)
    grid = (bn // _TILE,)
    emb, idx, loss = pl.pallas_call(
        functools.partial(_vq_kernel, inv_count=1.0 / float(bn * D)),
        grid=grid,
        in_specs=[
            pl.BlockSpec((_TILE, D), lambda i: (i, 0)),
            pl.BlockSpec((_M * _P, _DB), lambda i: (0, 0)),
        ],
        out_specs=(
            pl.BlockSpec((_TILE, D), lambda i: (i, 0)),
            pl.BlockSpec((_TILE, _M), lambda i: (i, 0)),
            pl.BlockSpec((1, 1), lambda i: (0, 0)),
        ),
        out_shape=(
            jax.ShapeDtypeStruct((bn, D), jnp.float32),
            jax.ShapeDtypeStruct((bn, _M), jnp.int32),
            jax.ShapeDtypeStruct((1, 1), jnp.float32),
        ),
        scratch_shapes=[
            pltpu.VMEM((3 * _DB, _M * _P), jnp.bfloat16),
            pltpu.VMEM((1, _M * _P), jnp.float32),
            pltpu.VMEM((_M * _P, _DB), jnp.bfloat16),
            pltpu.VMEM((_M * _P, _DB), jnp.bfloat16),
        ],
    )(flat_q, mem)
    embeddings = emb.reshape(B, N, D)
    indices = idx.reshape(B, N, _M)
    vq_loss = jnp.zeros((), jnp.float32)
    commitment_loss = loss.reshape(())
    return (embeddings, indices, vq_loss, commitment_loss)
